# baseline (speedup 1.0000x reference)
import jax
import jax.numpy as jnp
from jax import lax
from jax.experimental import pallas as pl
from jax.experimental.pallas import tpu as pltpu
from jax.experimental.pallas import tpu_sc as plsc

_B, _T, _NEOF, _EMB = 4, 2048, 128, 32
_NW = 32
_ROWS = (_B * _T) // _NW
_L = 16
_NJ = _NEOF // _L
_NK = _ROWS // _L
_NC = 4
_CROWS = _ROWS // _NC
_WOFF = 16


def _bcast(ref, idx):
    return plsc.load_gather(ref, [jnp.full((_L,), idx, jnp.int32)])


def _sc_body(x_hbm, tt_hbm, ft_hbm, w_hbm, b_hbm, out_hbm,
             x_v, tt_v, ft_v, w_v, tp_v, in_sems, out_sems):
    wid = lax.axis_index("s") * 2 + lax.axis_index("c")
    base = wid * _ROWS
    t0 = base % _T

    in_handles = [
        pltpu.async_copy(x_hbm.at[pl.ds(base + c * _CROWS, _CROWS)],
                         x_v.at[pl.ds(c * _CROWS, _CROWS)], in_sems[c])
        for c in range(_NC)
    ]
    pltpu.sync_copy(tt_hbm.at[pl.ds(t0, _ROWS)], tt_v)
    pltpu.sync_copy(ft_hbm, ft_v)
    pltpu.sync_copy(w_hbm, w_v.at[pl.ds(_WOFF, _EMB)])
    pltpu.sync_copy(b_hbm, w_v.at[pl.ds(8, 1)])

    iota = lax.iota(jnp.int32, _L)
    bb = _bcast(w_v, 8)
    facc = [bb for _ in range(_NJ)]
    tacc = [jnp.zeros((_L,), jnp.float32) for _ in range(_NK)]
    for e in range(_EMB):
        we = _bcast(w_v, _WOFF + e)
        col = jnp.full((_L,), e, jnp.int32)
        for j in range(_NJ):
            facc[j] = facc[j] + we * plsc.load_gather(
                ft_v, [iota + j * _L, col])
        for k in range(_NK):
            tacc[k] = tacc[k] + we * plsc.load_gather(
                tt_v, [iota + k * _L, col])
    for k in range(_NK):
        tp_v[pl.ds(k * _L, _L)] = tacc[k]

    out_handles = []
    for c in range(_NC):
        in_handles[c].wait()
        r0 = c * _CROWS

        def row_step(i, carry, r0=r0):
            r = r0 + i
            tpb = plsc.load_gather(tp_v, [jnp.full((_L,), r, jnp.int32)])
            for j in range(_NJ):
                sl = pl.ds(j * _L, _L)
                x_v[r, sl] = x_v[r, sl] + (facc[j] + tpb)
            return carry

        lax.fori_loop(0, _CROWS, row_step, 0)
        out_handles.append(
            pltpu.async_copy(x_v.at[pl.ds(r0, _CROWS)],
                             out_hbm.at[pl.ds(base + r0, _CROWS)],
                             out_sems[c]))
    for h in out_handles:
        h.wait()


def kernel(X, time_table, feature_table, W, b):
    Xf = X.reshape(_B * _T, _NEOF)

    mesh = plsc.VectorSubcoreMesh(core_axis_name="c", subcore_axis_name="s")
    run = pl.kernel(
        _sc_body,
        mesh=mesh,
        out_type=jax.ShapeDtypeStruct((_B * _T, _NEOF), jnp.float32),
        scratch_types=[
            pltpu.VMEM((_ROWS, _NEOF), jnp.float32),
            pltpu.VMEM((_ROWS, _EMB), jnp.float32),
            pltpu.VMEM((_NEOF, _EMB), jnp.float32),
            pltpu.VMEM((3 * _L,), jnp.float32),
            pltpu.VMEM((_ROWS,), jnp.float32),
            [pltpu.SemaphoreType.DMA] * _NC,
            [pltpu.SemaphoreType.DMA] * _NC,
        ],
        compiler_params=pltpu.CompilerParams(needs_layout_passes=False),
    )
    out = run(Xf, time_table, feature_table, W.reshape(_EMB), b)
    return out.reshape(_B, _T, _NEOF)


# speedup vs baseline: 1.0021x; 1.0021x over previous
import jax
import jax.numpy as jnp
from jax import lax
from jax.experimental import pallas as pl
from jax.experimental.pallas import tpu as pltpu
from jax.experimental.pallas import tpu_sc as plsc

_B, _T, _NEOF, _EMB = 4, 2048, 128, 32
_NW = 32
_ROWS = (_B * _T) // _NW
_L = 16
_NJ = _NEOF // _L
_NK = _ROWS // _L
_NC = 4
_CROWS = _ROWS // _NC
_WOFF = 16


def _bcast(ref, idx):
    return plsc.load_gather(ref, [jnp.full((_L,), idx, jnp.int32)])


def _sc_body(x_hbm, tt_hbm, ft_hbm, w_hbm, b_hbm, out_hbm,
             x_v, tt_v, ft_v, w_v, tp_v, in_sems, out_sems):
    wid = lax.axis_index("s") * 2 + lax.axis_index("c")
    base = wid * _ROWS
    t0 = base % _T

    with jax.named_scope("dma_issue"):
        in_handles = [
            pltpu.async_copy(x_hbm.at[pl.ds(base + c * _CROWS, _CROWS)],
                             x_v.at[pl.ds(c * _CROWS, _CROWS)], in_sems[c])
            for c in range(_NC)
        ]
    with jax.named_scope("small_dmas"):
        pltpu.sync_copy(tt_hbm.at[pl.ds(t0, _ROWS)], tt_v)
        pltpu.sync_copy(ft_hbm, ft_v)
        pltpu.sync_copy(w_hbm, w_v.at[pl.ds(_WOFF, _EMB)])
        pltpu.sync_copy(b_hbm, w_v.at[pl.ds(8, 1)])

    with jax.named_scope("proj_setup"):
        iota = lax.iota(jnp.int32, _L)
        bb = _bcast(w_v, 8)
        facc = [bb for _ in range(_NJ)]
        tacc = [jnp.zeros((_L,), jnp.float32) for _ in range(_NK)]
        for e in range(_EMB):
            we = _bcast(w_v, _WOFF + e)
            col = jnp.full((_L,), e, jnp.int32)
            for j in range(_NJ):
                facc[j] = facc[j] + we * plsc.load_gather(
                    ft_v, [iota + j * _L, col])
            for k in range(_NK):
                tacc[k] = tacc[k] + we * plsc.load_gather(
                    tt_v, [iota + k * _L, col])
        for k in range(_NK):
            tp_v[pl.ds(k * _L, _L)] = tacc[k]

    out_handles = []
    for c in range(_NC):
        with jax.named_scope(f"chunk{c}"):
            in_handles[c].wait()
            r0 = c * _CROWS

            def row_step(i, carry, r0=r0):
                r = r0 + i
                tpb = plsc.load_gather(tp_v, [jnp.full((_L,), r, jnp.int32)])
                for j in range(_NJ):
                    sl = pl.ds(j * _L, _L)
                    x_v[r, sl] = x_v[r, sl] + (facc[j] + tpb)
                return carry

            lax.fori_loop(0, _CROWS, row_step, 0)
            out_handles.append(
                pltpu.async_copy(x_v.at[pl.ds(r0, _CROWS)],
                                 out_hbm.at[pl.ds(base + r0, _CROWS)],
                                 out_sems[c]))
    with jax.named_scope("drain"):
        for h in out_handles:
            h.wait()


def kernel(X, time_table, feature_table, W, b):
    Xf = X.reshape(_B * _T, _NEOF)

    mesh = plsc.VectorSubcoreMesh(core_axis_name="c", subcore_axis_name="s")
    run = pl.kernel(
        _sc_body,
        mesh=mesh,
        out_type=jax.ShapeDtypeStruct((_B * _T, _NEOF), jnp.float32),
        scratch_types=[
            pltpu.VMEM((_ROWS, _NEOF), jnp.float32),
            pltpu.VMEM((_ROWS, _EMB), jnp.float32),
            pltpu.VMEM((_NEOF, _EMB), jnp.float32),
            pltpu.VMEM((3 * _L,), jnp.float32),
            pltpu.VMEM((_ROWS,), jnp.float32),
            [pltpu.SemaphoreType.DMA] * _NC,
            [pltpu.SemaphoreType.DMA] * _NC,
        ],
        compiler_params=pltpu.CompilerParams(needs_layout_passes=False),
    )
    out = run(Xf, time_table, feature_table, W.reshape(_EMB), b)
    return out.reshape(_B, _T, _NEOF)


# speedup vs baseline: 1.2343x; 1.2317x over previous
import jax
import jax.numpy as jnp
from jax import lax
from jax.experimental import pallas as pl
from jax.experimental.pallas import tpu as pltpu
from jax.experimental.pallas import tpu_sc as plsc

_B, _T, _NEOF, _EMB = 4, 2048, 128, 32
_NW = 32
_ROWS = (_B * _T) // _NW
_L = 16
_NJ = _NEOF // _L
_NC = 4
_CROWS = _ROWS // _NC


def _tc_proj(tt_ref, ft_ref, w_ref, b_ref, tp_ref, fb_ref):
    tp_ref[:] = jnp.sum(tt_ref[:] * w_ref[:], axis=1, keepdims=True)
    fb_ref[:] = jnp.sum(ft_ref[:] * w_ref[:], axis=1, keepdims=True) + b_ref[0]


def _sc_body(x_hbm, tp_hbm, fb_hbm, out_hbm, x_v, tp_v, fb_v,
             in_sems, out_sems):
    wid = lax.axis_index("s") * 2 + lax.axis_index("c")
    base = wid * _ROWS
    t0 = base % _T

    pltpu.sync_copy(tp_hbm.at[pl.ds(t0, _ROWS)], tp_v)
    pltpu.sync_copy(fb_hbm, fb_v)
    in_handles = [
        pltpu.async_copy(x_hbm.at[pl.ds(base + c * _CROWS, _CROWS)],
                         x_v.at[pl.ds(c * _CROWS, _CROWS)], in_sems[c])
        for c in range(_NC)
    ]
    facc = [fb_v[pl.ds(j * _L, _L)] for j in range(_NJ)]

    out_handles = []
    for c in range(_NC):
        in_handles[c].wait()
        r0 = c * _CROWS

        def row_step(i, carry, r0=r0):
            r = r0 + i
            tpb = plsc.load_gather(tp_v, [jnp.full((_L,), r, jnp.int32)])
            for j in range(_NJ):
                sl = pl.ds(j * _L, _L)
                x_v[r, sl] = x_v[r, sl] + (facc[j] + tpb)
            return carry

        lax.fori_loop(0, _CROWS, row_step, 0)
        out_handles.append(
            pltpu.async_copy(x_v.at[pl.ds(r0, _CROWS)],
                             out_hbm.at[pl.ds(base + r0, _CROWS)],
                             out_sems[c]))
    for h in out_handles:
        h.wait()


def kernel(X, time_table, feature_table, W, b):
    Xf = X.reshape(_B * _T, _NEOF)
    w_row = W.reshape(1, _EMB)

    tp2, fb2 = pl.pallas_call(
        _tc_proj,
        in_specs=[
            pl.BlockSpec((_T, _EMB), lambda: (0, 0)),
            pl.BlockSpec((_NEOF, _EMB), lambda: (0, 0)),
            pl.BlockSpec((1, _EMB), lambda: (0, 0)),
            pl.BlockSpec(memory_space=pltpu.SMEM),
        ],
        out_specs=[
            pl.BlockSpec((_T, 1), lambda: (0, 0)),
            pl.BlockSpec((_NEOF, 1), lambda: (0, 0)),
        ],
        out_shape=[
            jax.ShapeDtypeStruct((_T, 1), jnp.float32),
            jax.ShapeDtypeStruct((_NEOF, 1), jnp.float32),
        ],
    )(time_table, feature_table, w_row, b)

    mesh = plsc.VectorSubcoreMesh(core_axis_name="c", subcore_axis_name="s")
    run = pl.kernel(
        _sc_body,
        mesh=mesh,
        out_type=jax.ShapeDtypeStruct((_B * _T, _NEOF), jnp.float32),
        scratch_types=[
            pltpu.VMEM((_ROWS, _NEOF), jnp.float32),
            pltpu.VMEM((_ROWS,), jnp.float32),
            pltpu.VMEM((_NEOF,), jnp.float32),
            [pltpu.SemaphoreType.DMA] * _NC,
            [pltpu.SemaphoreType.DMA] * _NC,
        ],
        compiler_params=pltpu.CompilerParams(needs_layout_passes=False),
    )
    out = run(Xf, tp2.reshape(_T), fb2.reshape(_NEOF))
    return out.reshape(_B, _T, _NEOF)


# speedup vs baseline: 1.3599x; 1.1017x over previous
import jax
import jax.numpy as jnp
from jax import lax
from jax.experimental import pallas as pl
from jax.experimental.pallas import tpu as pltpu
from jax.experimental.pallas import tpu_sc as plsc

_B, _T, _NEOF, _EMB = 4, 2048, 128, 32
_NW = 32
_ROWS = (_B * _T) // _NW
_L = 16
_NJ = _NEOF // _L
_NC = 4
_CROWS = _ROWS // _NC


def _tc_proj(tt_ref, ft_ref, w_ref, b_ref, tp_ref, fb_ref):
    tp = jnp.sum(tt_ref[:] * w_ref[:], axis=1)
    fb = jnp.sum(ft_ref[:] * w_ref[:], axis=1) + b_ref[0]
    tp_ref[:] = tp.reshape(_T // _NEOF, _NEOF)
    fb_ref[:] = fb.reshape(1, _NEOF)


def _sc_body(x_hbm, tp_hbm, fb_hbm, out_hbm, x_v, tp_v, fb_v,
             in_sems, out_sems):
    wid = lax.axis_index("s") * 2 + lax.axis_index("c")
    base = wid * _ROWS
    t0 = base % _T

    pltpu.sync_copy(tp_hbm, tp_v)
    pltpu.sync_copy(fb_hbm, fb_v)
    in_handles = [
        pltpu.async_copy(x_hbm.at[pl.ds(base + c * _CROWS, _CROWS)],
                         x_v.at[pl.ds(c * _CROWS, _CROWS)], in_sems[c])
        for c in range(_NC)
    ]
    facc = [fb_v[0, pl.ds(j * _L, _L)] for j in range(_NJ)]

    out_handles = []
    for c in range(_NC):
        in_handles[c].wait()
        r0 = c * _CROWS

        def row_step(i, carry, r0=r0):
            r = r0 + i
            t = t0 + r
            tpb = plsc.load_gather(
                tp_v, [jnp.full((_L,), t >> 7, jnp.int32),
                       jnp.full((_L,), t & (_NEOF - 1), jnp.int32)])
            for j in range(_NJ):
                sl = pl.ds(j * _L, _L)
                x_v[r, sl] = x_v[r, sl] + (facc[j] + tpb)
            return carry

        lax.fori_loop(0, _CROWS, row_step, 0)
        out_handles.append(
            pltpu.async_copy(x_v.at[pl.ds(r0, _CROWS)],
                             out_hbm.at[pl.ds(base + r0, _CROWS)],
                             out_sems[c]))
    for h in out_handles:
        h.wait()


def kernel(X, time_table, feature_table, W, b):
    Xf = X.reshape(_B * _T, _NEOF)
    w_row = W.reshape(1, _EMB)

    tp2, fb2 = pl.pallas_call(
        _tc_proj,
        in_specs=[
            pl.BlockSpec((_T, _EMB), lambda: (0, 0)),
            pl.BlockSpec((_NEOF, _EMB), lambda: (0, 0)),
            pl.BlockSpec((1, _EMB), lambda: (0, 0)),
            pl.BlockSpec(memory_space=pltpu.SMEM),
        ],
        out_specs=[
            pl.BlockSpec((_T // _NEOF, _NEOF), lambda: (0, 0)),
            pl.BlockSpec((1, _NEOF), lambda: (0, 0)),
        ],
        out_shape=[
            jax.ShapeDtypeStruct((_T // _NEOF, _NEOF), jnp.float32),
            jax.ShapeDtypeStruct((1, _NEOF), jnp.float32),
        ],
    )(time_table, feature_table, w_row, b)

    mesh = plsc.VectorSubcoreMesh(core_axis_name="c", subcore_axis_name="s")
    run = pl.kernel(
        _sc_body,
        mesh=mesh,
        out_type=jax.ShapeDtypeStruct((_B * _T, _NEOF), jnp.float32),
        scratch_types=[
            pltpu.VMEM((_ROWS, _NEOF), jnp.float32),
            pltpu.VMEM((_T // _NEOF, _NEOF), jnp.float32),
            pltpu.VMEM((1, _NEOF), jnp.float32),
            [pltpu.SemaphoreType.DMA] * _NC,
            [pltpu.SemaphoreType.DMA] * _NC,
        ],
        compiler_params=pltpu.CompilerParams(
            needs_layout_passes=False, use_tc_tiling_on_sc=True),
    )
    out = run(Xf, tp2, fb2)
    return out.reshape(_B, _T, _NEOF)
